# baseline (speedup 1.0000x reference)
import functools

import jax
import jax.numpy as jnp
from jax import lax
from jax.experimental import pallas as pl
from jax.experimental.pallas import tpu as pltpu
from jax.experimental.pallas import tpu_sc as plsc

D = 32
NC = 2
NS = 16
NW = NC * NS
CH = 1280


def _make_gather(total):
    bpw = total // NW
    nchunk = bpw // CH
    mesh = plsc.VectorSubcoreMesh(core_axis_name="c", subcore_axis_name="s")

    @functools.partial(
        pl.kernel,
        mesh=mesh,
        out_type=jax.ShapeDtypeStruct((total, D), jnp.float32),
        scratch_types=[
            pltpu.VMEM((CH,), jnp.int32),
            pltpu.VMEM((CH,), jnp.int32),
            pltpu.VMEM((CH, D), jnp.float32),
            pltpu.VMEM((CH, D), jnp.float32),
            pltpu.SemaphoreType.DMA,
            pltpu.SemaphoreType.DMA,
            pltpu.SemaphoreType.DMA,
        ],
        compiler_params=pltpu.CompilerParams(use_tc_tiling_on_sc=False),
    )
    def gather_kernel(idx_hbm, table_hbm, out_hbm,
                      idx_v0, idx_v1, rows_v0, rows_v1, gsem, ssem0, ssem1):
        wid = lax.axis_index("s") * NC + lax.axis_index("c")
        base = wid * bpw
        idx_vs = (idx_v0, idx_v1)
        rows_vs = (rows_v0, rows_v1)
        ssems = (ssem0, ssem1)
        gathers = [None, None]
        stores = [None, None]
        pltpu.sync_copy(idx_hbm.at[pl.ds(base, CH)], idx_v0)
        gathers[0] = pltpu.async_copy(table_hbm.at[idx_v0], rows_v0, gsem)
        if nchunk > 1:
            pltpu.sync_copy(idx_hbm.at[pl.ds(base + CH, CH)], idx_v1)
        for c in range(nchunk):
            b = c % 2
            gathers[b].wait()
            stores[b] = pltpu.async_copy(
                rows_vs[b], out_hbm.at[pl.ds(base + c * CH, CH)], ssems[b])
            if c + 1 < nchunk:
                nb_ = 1 - b
                if stores[nb_] is not None:
                    stores[nb_].wait()
                gathers[nb_] = pltpu.async_copy(
                    table_hbm.at[idx_vs[nb_]], rows_vs[nb_], gsem)
                if c + 2 < nchunk:
                    pltpu.sync_copy(
                        idx_hbm.at[pl.ds(base + (c + 2) * CH, CH)], idx_vs[b])
        if nchunk > 1:
            stores[(nchunk - 2) % 2].wait()
        stores[(nchunk - 1) % 2].wait()

    return gather_kernel


def kernel(indices, weight):
    nb, nh = indices.shape
    flat = indices.T.reshape(-1).astype(jnp.int32)
    half = flat.shape[0] // 2
    g = _make_gather(half)
    o1 = g(flat[:half], weight)
    o2 = g(flat[half:], weight)
    h2 = nh // 2
    out = jnp.concatenate(
        [o1.reshape(h2, nb, D), o2.reshape(nh - h2, nb, D)], axis=0)
    return out.transpose(1, 0, 2)


# speedup vs baseline: 1.0359x; 1.0359x over previous
import functools

import jax
import jax.numpy as jnp
from jax import lax
from jax.experimental import pallas as pl
from jax.experimental.pallas import tpu as pltpu
from jax.experimental.pallas import tpu_sc as plsc

D = 32
NC = 2
NS = 16
NW = NC * NS
CH = 1280


def _make_gather(total):
    bpw = total // NW
    nchunk = bpw // CH
    mesh = plsc.VectorSubcoreMesh(core_axis_name="c", subcore_axis_name="s")

    @functools.partial(
        pl.kernel,
        mesh=mesh,
        out_type=jax.ShapeDtypeStruct((total, D), jnp.float32),
        scratch_types=[
            pltpu.VMEM((CH,), jnp.int32),
            pltpu.VMEM((CH,), jnp.int32),
            pltpu.VMEM((CH, D), jnp.float32),
            pltpu.VMEM((CH, D), jnp.float32),
            pltpu.SemaphoreType.DMA,
            pltpu.SemaphoreType.DMA,
            pltpu.SemaphoreType.DMA,
        ],
        compiler_params=pltpu.CompilerParams(use_tc_tiling_on_sc=False),
    )
    def gather_kernel(idx_hbm, table_hbm, out_hbm,
                      idx_v0, idx_v1, rows_v0, rows_v1, gsem, ssem0, ssem1):
        wid = lax.axis_index("s") * NC + lax.axis_index("c")
        base = wid * bpw
        idx_vs = (idx_v0, idx_v1)
        rows_vs = (rows_v0, rows_v1)
        ssems = (ssem0, ssem1)
        gathers = [None, None]
        stores = [None, None]
        pltpu.sync_copy(idx_hbm.at[pl.ds(base, CH)], idx_v0)
        gathers[0] = pltpu.async_copy(table_hbm.at[idx_v0], rows_v0, gsem)
        if nchunk > 1:
            pltpu.sync_copy(idx_hbm.at[pl.ds(base + CH, CH)], idx_v1)
        for c in range(nchunk):
            b = c % 2
            gathers[b].wait()
            stores[b] = pltpu.async_copy(
                rows_vs[b], out_hbm.at[pl.ds(base + c * CH, CH)], ssems[b])
            if c + 1 < nchunk:
                nb_ = 1 - b
                if stores[nb_] is not None:
                    stores[nb_].wait()
                gathers[nb_] = pltpu.async_copy(
                    table_hbm.at[idx_vs[nb_]], rows_vs[nb_], gsem)
                if c + 2 < nchunk:
                    pltpu.sync_copy(
                        idx_hbm.at[pl.ds(base + (c + 2) * CH, CH)], idx_vs[b])
        if nchunk > 1:
            stores[(nchunk - 2) % 2].wait()
        stores[(nchunk - 1) % 2].wait()

    return gather_kernel


def kernel(indices, weight):
    nb, nh = indices.shape
    flat = indices.T.reshape(-1).astype(jnp.int32)
    out = _make_gather(flat.shape[0])(flat, weight)
    return out.reshape(nh, nb, weight.shape[1]).transpose(1, 0, 2)
